# baseline (speedup 1.0000x reference)
import functools

import jax
import jax.numpy as jnp
from jax import lax
from jax.experimental import pallas as pl
from jax.experimental.pallas import tpu as pltpu
from jax.experimental.pallas import tpu_sc as plsc

D = 128
NC = 2
NS = 16
NW = NC * NS


def kernel(pos_enc_1D, pos):
    B, S = pos.shape
    V = pos_enc_1D.shape[0]
    N = B * S
    per_w = N // NW
    CH = 256
    n_ch = per_w // CH
    NB = 2
    G = CH // 16

    idx_flat = pos.reshape(N)
    tbl_1d = pos_enc_1D.reshape(V * D)
    mesh = plsc.VectorSubcoreMesh(core_axis_name="c", subcore_axis_name="s")

    @functools.partial(
        pl.kernel,
        mesh=mesh,
        compiler_params=pltpu.CompilerParams(needs_layout_passes=False),
        out_type=jax.ShapeDtypeStruct((N * D,), jnp.float32),
        scratch_types=[
            pltpu.VMEM((per_w,), jnp.int32),
            pltpu.VMEM((CH * D,), jnp.float32),
            pltpu.VMEM((CH * D,), jnp.float32),
            pltpu.VMEM((V * D,), jnp.float32),
            pltpu.SemaphoreType.DMA,
            pltpu.SemaphoreType.DMA,
            pltpu.SemaphoreType.DMA,
        ],
    )
    def gather_kernel(tbl_hbm, idx_hbm, out_hbm, idx_v, rows0, rows1,
                      table_v, sem_i, sem_o0, sem_o1):
        cid = lax.axis_index("c")
        sid = lax.axis_index("s")
        wid = sid * NC + cid
        base = wid * per_w

        idx_cp = pltpu.make_async_copy(
            idx_hbm.at[pl.ds(base, per_w)], idx_v, sem_i)
        idx_cp.start()
        pltpu.sync_copy(tbl_hbm, table_v)
        idx_cp.wait()

        bufs = ((rows0, sem_o0), (rows1, sem_o1))
        iota16 = lax.iota(jnp.int32, 16)
        lanes = [jnp.full((16,), r, jnp.int32) for r in range(16)]

        def start_out(i, rows, sem):
            pltpu.make_async_copy(
                rows, out_hbm.at[pl.ds((base + i * CH) * D, CH * D)],
                sem).start()

        def wait_out(rows, sem):
            pltpu.make_async_copy(
                rows, out_hbm.at[pl.ds(base * D, CH * D)], sem).wait()

        def build_chunk(i, rows):
            def grp(g, carry):
                idx16 = idx_v[pl.ds(i * CH + g * 16, 16)]
                for r in range(16):
                    addr0 = idx16.at[lanes[r]].get(
                        mode="promise_in_bounds") * D
                    roff = (g * 16 + r) * D
                    for j in range(8):
                        vals = plsc.load_gather(
                            table_v, [addr0 + (iota16 + 16 * j)])
                        rows[pl.ds(roff + 16 * j, 16)] = vals
                return carry

            lax.fori_loop(0, G, grp, 0)

        def body(k, carry):
            i0 = NB * k
            for b in range(NB):
                rows, sem_o = bufs[b]
                i = i0 + b

                @pl.when(i >= NB)
                def _():
                    wait_out(rows, sem_o)

                build_chunk(i, rows)
                start_out(i, rows, sem_o)
            return carry

        lax.fori_loop(0, n_ch // NB, body, 0)

        for b in range(NB):
            wait_out(bufs[b][0], bufs[b][1])

    out = gather_kernel(tbl_1d, idx_flat)
    return out.reshape(B, S, D)


# speedup vs baseline: 4.0111x; 4.0111x over previous
import functools

import jax
import jax.numpy as jnp
from jax import lax
from jax.experimental import pallas as pl
from jax.experimental.pallas import tpu as pltpu
from jax.experimental.pallas import tpu_sc as plsc

D = 128
NC = 2
NS = 16
NW = NC * NS
NR = 4


def kernel(pos_enc_1D, pos):
    B, S = pos.shape
    V = pos_enc_1D.shape[0]
    N = B * S
    per_w = N // NW
    CH = 256
    n_ch = per_w // CH
    NB = 3

    idx_flat = pos.reshape(N)
    mesh = plsc.VectorSubcoreMesh(core_axis_name="c", subcore_axis_name="s")

    @functools.partial(
        pl.kernel,
        mesh=mesh,
        out_type=jax.ShapeDtypeStruct((N, D), jnp.float32),
        scratch_types=[
            pltpu.VMEM((per_w,), jnp.int32),
            pltpu.VMEM((CH, D), jnp.float32),
            pltpu.VMEM((CH, D), jnp.float32),
            pltpu.VMEM((CH, D), jnp.float32),
            pltpu.VMEM_SHARED((NR * V, D), jnp.float32),
            pltpu.SemaphoreType.DMA,
            pltpu.SemaphoreType.DMA,
            pltpu.SemaphoreType.DMA,
            pltpu.SemaphoreType.DMA,
        ],
    )
    def gather_kernel(table_hbm, idx_hbm, out_hbm, idx_v, rows0, rows1,
                      rows2, table_sp, sem_i, sem_g0, sem_g1, sem_g2):
        cid = lax.axis_index("c")
        sid = lax.axis_index("s")
        wid = sid * NC + cid
        base = wid * per_w

        idx_cp = pltpu.make_async_copy(
            idx_hbm.at[pl.ds(base, per_w)], idx_v, sem_i)
        idx_cp.start()

        rep = sid % NR

        @pl.when(sid < NR)
        def _():
            pltpu.sync_copy(table_hbm, table_sp.at[pl.ds(sid * V, V)])

        idx_cp.wait()
        bias = rep * V

        def adj(t, carry):
            off = t * 64
            for u in range(4):
                sl = pl.ds(off + u * 16, 16)
                idx_v[sl] = idx_v[sl] + bias
            return carry

        lax.fori_loop(0, per_w // 64, adj, 0)
        plsc.subcore_barrier()

        bufs = ((rows0, sem_g0), (rows1, sem_g1), (rows2, sem_g2))

        def start_gather(i, rows, sem):
            pltpu.make_async_copy(
                table_sp.at[idx_v.at[pl.ds(i * CH, CH)]], rows, sem).start()

        def wait_gather(rows, sem):
            pltpu.make_async_copy(
                table_sp.at[idx_v.at[pl.ds(0, CH)]], rows, sem).wait()

        for k in range(NB - 1):
            start_gather(k, *bufs[k])

        def body(j, carry):
            i0 = NB * j
            for k in range(NB):
                rows, sem = bufs[k]
                nxt = i0 + k + NB - 1

                @pl.when(nxt < n_ch)
                def _():
                    nrows, nsem = bufs[(k + NB - 1) % NB]
                    start_gather(nxt, nrows, nsem)

                wait_gather(rows, sem)
                pltpu.sync_copy(
                    rows, out_hbm.at[pl.ds(base + (i0 + k) * CH, CH)])
            return carry

        lax.fori_loop(0, n_ch // NB, body, 0)

        for i in range(NB * (n_ch // NB), n_ch):
            rows, sem = bufs[i % NB]
            wait_gather(rows, sem)
            pltpu.sync_copy(rows, out_hbm.at[pl.ds(base + i * CH, CH)])

    out = gather_kernel(pos_enc_1D, idx_flat)
    return out.reshape(B, S, D)


# speedup vs baseline: 4.0588x; 1.0119x over previous
import functools

import jax
import jax.numpy as jnp
from jax import lax
from jax.experimental import pallas as pl
from jax.experimental.pallas import tpu as pltpu
from jax.experimental.pallas import tpu_sc as plsc

D = 128
NC = 2
NS = 16
NW = NC * NS


def kernel(pos_enc_1D, pos):
    B, S = pos.shape
    V = pos_enc_1D.shape[0]
    N = B * S
    per_w = N // NW
    CH = 256
    n_ch = per_w // CH
    NB = 3

    idx_flat = pos.reshape(N)
    mesh = plsc.VectorSubcoreMesh(core_axis_name="c", subcore_axis_name="s")

    @functools.partial(
        pl.kernel,
        mesh=mesh,
        out_type=jax.ShapeDtypeStruct((N, D), jnp.float32),
        scratch_types=[
            pltpu.VMEM((per_w,), jnp.int32),
            pltpu.VMEM((CH, D), jnp.float32),
            pltpu.VMEM((CH, D), jnp.float32),
            pltpu.VMEM((CH, D), jnp.float32),
            pltpu.VMEM_SHARED((V, D), jnp.float32),
            pltpu.SemaphoreType.DMA,
            pltpu.SemaphoreType.DMA,
            pltpu.SemaphoreType.DMA,
            pltpu.SemaphoreType.DMA,
        ],
    )
    def gather_kernel(table_hbm, idx_hbm, out_hbm, idx_v, rows0, rows1,
                      rows2, table_sp, sem_i, sem_g0, sem_g1, sem_g2):
        cid = lax.axis_index("c")
        sid = lax.axis_index("s")
        wid = sid * NC + cid
        base = wid * per_w

        idx_cp = pltpu.make_async_copy(
            idx_hbm.at[pl.ds(base, per_w)], idx_v, sem_i)
        idx_cp.start()

        @pl.when(sid == 0)
        def _():
            pltpu.sync_copy(table_hbm, table_sp)

        plsc.subcore_barrier()
        idx_cp.wait()

        bufs = ((rows0, sem_g0), (rows1, sem_g1), (rows2, sem_g2))

        def start_gather(i, rows, sem):
            pltpu.make_async_copy(
                table_sp.at[idx_v.at[pl.ds(i * CH, CH)]], rows, sem).start()

        def wait_gather(rows, sem):
            pltpu.make_async_copy(
                table_sp.at[idx_v.at[pl.ds(0, CH)]], rows, sem).wait()

        for k in range(NB - 1):
            start_gather(k, *bufs[k])

        def body(j, carry):
            i0 = NB * j
            for k in range(NB):
                rows, sem = bufs[k]
                nxt = i0 + k + NB - 1

                @pl.when(nxt < n_ch)
                def _():
                    nrows, nsem = bufs[(k + NB - 1) % NB]
                    start_gather(nxt, nrows, nsem)

                wait_gather(rows, sem)
                pltpu.sync_copy(
                    rows, out_hbm.at[pl.ds(base + (i0 + k) * CH, CH)])
            return carry

        lax.fori_loop(0, n_ch // NB, body, 0)

        for i in range(NB * (n_ch // NB), n_ch):
            rows, sem = bufs[i % NB]
            wait_gather(rows, sem)
            pltpu.sync_copy(rows, out_hbm.at[pl.ds(base + i * CH, CH)])

    out = gather_kernel(pos_enc_1D, idx_flat)
    return out.reshape(B, S, D)
